# baseline (speedup 1.0000x reference)
import jax
import jax.numpy as jnp
from jax.experimental import pallas as pl
from jax.experimental.pallas import tpu as pltpu

DIM = 6144
D3 = DIM // 3
B, S = 4, 2048
BS = 256


def _add_body(idx_ref, x_ref, q_ref, k_ref, v_ref, o_ref):
    del idx_ref
    xr = x_ref[0]
    o_ref[0, :, 0 * D3:1 * D3] = xr[:, 0 * D3:1 * D3] + q_ref[0]
    o_ref[0, :, 1 * D3:2 * D3] = xr[:, 1 * D3:2 * D3] + k_ref[0]
    o_ref[0, :, 2 * D3:3 * D3] = xr[:, 2 * D3:3 * D3] + v_ref[0]


def kernel(x, bias_idx, q_bias, k_bias, v_bias):
    idx = bias_idx.astype(jnp.int32)
    n = q_bias.shape[0]
    qb = q_bias.reshape(n, 1, D3)
    kb = k_bias.reshape(n, 1, D3)
    vb = v_bias.reshape(n, 1, D3)
    grid = (B, S // BS)
    return pl.pallas_call(
        _add_body,
        grid_spec=pltpu.PrefetchScalarGridSpec(
            num_scalar_prefetch=1,
            grid=grid,
            in_specs=[
                pl.BlockSpec((1, BS, DIM), lambda b, s, i: (b, s, 0)),
                pl.BlockSpec((1, 1, D3), lambda b, s, i: (i[b], 0, 0)),
                pl.BlockSpec((1, 1, D3), lambda b, s, i: (i[b], 0, 0)),
                pl.BlockSpec((1, 1, D3), lambda b, s, i: (i[b], 0, 0)),
            ],
            out_specs=pl.BlockSpec((1, BS, DIM), lambda b, s, i: (b, s, 0)),
        ),
        out_shape=jax.ShapeDtypeStruct((B, S, DIM), jnp.float32),
        compiler_params=pltpu.CompilerParams(
            dimension_semantics=("parallel", "arbitrary"),
        ),
    )(idx, x, qb, kb, vb)


# speedup vs baseline: 1.0053x; 1.0053x over previous
import jax
import jax.numpy as jnp
from jax.experimental import pallas as pl
from jax.experimental.pallas import tpu as pltpu

DIM = 6144
D3 = DIM // 3
B, S = 4, 2048
BS = 512


def _add_body(idx_ref, x_ref, q_ref, k_ref, v_ref, o_ref):
    del idx_ref
    xr = x_ref[0]
    o_ref[0, :, 0 * D3:1 * D3] = xr[:, 0 * D3:1 * D3] + q_ref[0]
    o_ref[0, :, 1 * D3:2 * D3] = xr[:, 1 * D3:2 * D3] + k_ref[0]
    o_ref[0, :, 2 * D3:3 * D3] = xr[:, 2 * D3:3 * D3] + v_ref[0]


def kernel(x, bias_idx, q_bias, k_bias, v_bias):
    idx = bias_idx.astype(jnp.int32)
    n = q_bias.shape[0]
    qb = q_bias.reshape(n, 1, D3)
    kb = k_bias.reshape(n, 1, D3)
    vb = v_bias.reshape(n, 1, D3)
    grid = (B, S // BS)
    return pl.pallas_call(
        _add_body,
        grid_spec=pltpu.PrefetchScalarGridSpec(
            num_scalar_prefetch=1,
            grid=grid,
            in_specs=[
                pl.BlockSpec((1, BS, DIM), lambda b, s, i: (b, s, 0)),
                pl.BlockSpec((1, 1, D3), lambda b, s, i: (i[b], 0, 0)),
                pl.BlockSpec((1, 1, D3), lambda b, s, i: (i[b], 0, 0)),
                pl.BlockSpec((1, 1, D3), lambda b, s, i: (i[b], 0, 0)),
            ],
            out_specs=pl.BlockSpec((1, BS, DIM), lambda b, s, i: (b, s, 0)),
        ),
        out_shape=jax.ShapeDtypeStruct((B, S, DIM), jnp.float32),
        compiler_params=pltpu.CompilerParams(
            dimension_semantics=("parallel", "arbitrary"),
        ),
    )(idx, x, qb, kb, vb)


# speedup vs baseline: 1.1865x; 1.1802x over previous
import jax
import jax.numpy as jnp
from jax.experimental import pallas as pl
from jax.experimental.pallas import tpu as pltpu

DIM = 6144
D3 = DIM // 3
B, S = 4, 2048
BS = 512


def _add_body(idx_ref, x_ref, q_hbm, k_hbm, v_hbm, o_ref,
              qs, ks, vs, sem):
    b = pl.program_id(0)
    s = pl.program_id(1)

    @pl.when(jnp.logical_and(b == 0, s == 0))
    def _fetch_bias():
        copies = []
        for bb in range(B):
            i = idx_ref[bb]
            for tab, dst in ((q_hbm, qs), (k_hbm, ks), (v_hbm, vs)):
                cp = pltpu.make_async_copy(
                    tab.at[pl.ds(i, 1), :], dst.at[pl.ds(bb, 1), :], sem)
                cp.start()
                copies.append(cp)
        for cp in copies:
            cp.wait()

    xr = x_ref[0]
    o_ref[0, :, 0 * D3:1 * D3] = xr[:, 0 * D3:1 * D3] + qs[pl.ds(b, 1), :]
    o_ref[0, :, 1 * D3:2 * D3] = xr[:, 1 * D3:2 * D3] + ks[pl.ds(b, 1), :]
    o_ref[0, :, 2 * D3:3 * D3] = xr[:, 2 * D3:3 * D3] + vs[pl.ds(b, 1), :]


def kernel(x, bias_idx, q_bias, k_bias, v_bias):
    idx = bias_idx.astype(jnp.int32)
    grid = (B, S // BS)
    return pl.pallas_call(
        _add_body,
        grid_spec=pltpu.PrefetchScalarGridSpec(
            num_scalar_prefetch=1,
            grid=grid,
            in_specs=[
                pl.BlockSpec((1, BS, DIM), lambda b, s, i: (b, s, 0)),
                pl.BlockSpec(memory_space=pl.ANY),
                pl.BlockSpec(memory_space=pl.ANY),
                pl.BlockSpec(memory_space=pl.ANY),
            ],
            out_specs=pl.BlockSpec((1, BS, DIM), lambda b, s, i: (b, s, 0)),
            scratch_shapes=[
                pltpu.VMEM((B, D3), jnp.float32),
                pltpu.VMEM((B, D3), jnp.float32),
                pltpu.VMEM((B, D3), jnp.float32),
                pltpu.SemaphoreType.DMA,
            ],
        ),
        out_shape=jax.ShapeDtypeStruct((B, S, DIM), jnp.float32),
        compiler_params=pltpu.CompilerParams(
            dimension_semantics=("arbitrary", "arbitrary"),
        ),
    )(idx, x, q_bias, k_bias, v_bias)
